# baseline (speedup 1.0000x reference)
import functools

import jax
import jax.numpy as jnp
from jax import lax
from jax.experimental import pallas as pl
from jax.experimental.pallas import tpu as pltpu
from jax.experimental.pallas import tpu_sc as plsc

SQ, B, NH, DIM = 8192, 2, 16, 128
MAX_SEQ = 8192
HALF = DIM // 2
ROWS = SQ * B
TOT = 2 * ROWS
NC, NS = 2, 16
NW = NC * NS
RPW = TOT // NW
CHUNK = 128
NCHUNK = RPW // CHUNK


def _sc_gather(tab, pid2d):
    hc = ROWS // CHUNK // NW
    mesh = plsc.VectorSubcoreMesh(core_axis_name="c", subcore_axis_name="s")

    @functools.partial(
        pl.kernel,
        out_type=jax.ShapeDtypeStruct((ROWS, DIM), jnp.float32),
        mesh=mesh,
        scratch_types=[
            pltpu.VMEM((hc, CHUNK), jnp.int32),
            pltpu.VMEM((CHUNK, DIM), jnp.float32),
            pltpu.VMEM((CHUNK, DIM), jnp.float32),
            pltpu.SemaphoreType.DMA,
            pltpu.SemaphoreType.DMA,
        ],
    )
    def gather_kernel(tab_hbm, pid_hbm, out_hbm, idx_v, buf0, buf1, sem0, sem1):
        wid = lax.axis_index("s") * NC + lax.axis_index("c")
        bufs = (buf0, buf1)
        sems = (sem0, sem1)

        pltpu.sync_copy(pid_hbm.at[pl.ds(wid * hc, hc)], idx_v)

        cp = pltpu.async_copy(tab_hbm.at[idx_v.at[0]], bufs[0], sems[0])
        for j in range(hc):
            nxt = None
            if j + 1 < hc:
                nxt = pltpu.async_copy(
                    tab_hbm.at[idx_v.at[j + 1]], bufs[(j + 1) % 2],
                    sems[(j + 1) % 2])
            cp.wait()
            pltpu.sync_copy(
                bufs[j % 2],
                out_hbm.at[pl.ds(wid * (hc * CHUNK) + j * CHUNK, CHUNK)])
            cp = nxt

    return gather_kernel(tab, pid2d)


S_BLK = 256


def _expansion_matrix():
    r = lax.broadcasted_iota(jnp.int32, (DIM, 2 * DIM), 0)
    c = lax.broadcasted_iota(jnp.int32, (DIM, 2 * DIM), 1)
    cond_a = (r < HALF) & (c < DIM) & ((c >> 1) == r)
    cond_b = (r >= HALF) & (c >= DIM) & (((c - DIM) >> 1) == (r - HALF))
    sign = jnp.where((c & 1) == 1, 1.0, -1.0).astype(jnp.float32)
    return (jnp.where(cond_a, 1.0, 0.0) +
            jnp.where(cond_b, sign, 0.0)).astype(jnp.float32)


def _tc_apply(x, f2):
    def body(x_ref, f_ref, o_ref):
        lane = lax.broadcasted_iota(jnp.int32, (S_BLK, B, NH, DIM), 3)
        swap_idx = lane ^ 1
        e_mat = _expansion_matrix()
        x_v = x_ref[...]
        ab = lax.dot_general(f_ref[...], e_mat, (((1,), (0,)), ((), ())),
                             precision=lax.Precision.HIGHEST,
                             preferred_element_type=jnp.float32)
        a = ab[:, :DIM].reshape(S_BLK, B, 1, DIM)
        b = ab[:, DIM:].reshape(S_BLK, B, 1, DIM)
        xs = jnp.take_along_axis(x_v, swap_idx, axis=3)
        o_ref[...] = x_v * a + xs * b

    x_spec = pl.BlockSpec((S_BLK, B, NH, DIM), lambda i: (i, 0, 0, 0))
    f_spec = pl.BlockSpec((S_BLK * B, DIM), lambda i: (i, 0))
    return pl.pallas_call(
        body,
        grid=(SQ // S_BLK,),
        in_specs=[x_spec, f_spec],
        out_specs=x_spec,
        out_shape=jax.ShapeDtypeStruct((SQ, B, NH, DIM), jnp.float32),
        compiler_params=pltpu.CompilerParams(
            dimension_semantics=("parallel",),
            vmem_limit_bytes=120 * 1024 * 1024),
    )(x, f2)


def kernel(query, key, query_position_ids, key_position_ids,
           freqs_cis_real, freqs_cis_imag):
    tab = jnp.concatenate([freqs_cis_real.astype(jnp.float32),
                           freqs_cis_imag.astype(jnp.float32)], axis=1)

    qpid2d = query_position_ids.astype(jnp.int32).reshape(ROWS // CHUNK, CHUNK)
    kpid2d = key_position_ids.astype(jnp.int32).reshape(ROWS // CHUNK, CHUNK)

    fq = _sc_gather(tab, qpid2d)
    fk = _sc_gather(tab, kpid2d)
    q_out = _tc_apply(query.astype(jnp.float32), fq)
    k_out = _tc_apply(key.astype(jnp.float32), fk)
    return q_out, k_out


# speedup vs baseline: 1.0243x; 1.0243x over previous
import functools

import jax
import jax.numpy as jnp
from jax import lax
from jax.experimental import pallas as pl
from jax.experimental.pallas import tpu as pltpu
from jax.experimental.pallas import tpu_sc as plsc

SQ, B, NH, DIM = 8192, 2, 16, 128
MAX_SEQ = 8192
HALF = DIM // 2
ROWS = SQ * B
TOT = 2 * ROWS
NC, NS = 2, 16
NW = NC * NS
RPW = TOT // NW
CHUNK = 128
NCHUNK = RPW // CHUNK


def _sc_gather(tab, qpid2d, kpid2d):
    hc = NCHUNK // 2
    mesh = plsc.VectorSubcoreMesh(core_axis_name="c", subcore_axis_name="s")

    @functools.partial(
        pl.kernel,
        out_type=jax.ShapeDtypeStruct((TOT, DIM), jnp.float32),
        mesh=mesh,
        scratch_types=[
            pltpu.VMEM((NCHUNK, CHUNK), jnp.int32),
            pltpu.VMEM((CHUNK, DIM), jnp.float32),
            pltpu.VMEM((CHUNK, DIM), jnp.float32),
            pltpu.SemaphoreType.DMA,
            pltpu.SemaphoreType.DMA,
        ],
    )
    def gather_kernel(tab_hbm, qpid_hbm, kpid_hbm, out_hbm,
                      idx_v, buf0, buf1, sem0, sem1):
        wid = lax.axis_index("s") * NC + lax.axis_index("c")
        bufs = (buf0, buf1)
        sems = (sem0, sem1)

        pltpu.sync_copy(qpid_hbm.at[pl.ds(wid * hc, hc)], idx_v.at[pl.ds(0, hc)])
        pltpu.sync_copy(kpid_hbm.at[pl.ds(wid * hc, hc)],
                        idx_v.at[pl.ds(hc, hc)])

        def out_off(j):
            if j < hc:
                return wid * (hc * CHUNK) + j * CHUNK
            return ROWS + wid * (hc * CHUNK) + (j - hc) * CHUNK

        cp = pltpu.async_copy(tab_hbm.at[idx_v.at[0]], bufs[0], sems[0])
        for j in range(NCHUNK):
            nxt = None
            if j + 1 < NCHUNK:
                nxt = pltpu.async_copy(
                    tab_hbm.at[idx_v.at[j + 1]], bufs[(j + 1) % 2],
                    sems[(j + 1) % 2])
            cp.wait()
            pltpu.sync_copy(
                bufs[j % 2],
                out_hbm.at[pl.ds(out_off(j), CHUNK)])
            cp = nxt

    return gather_kernel(tab, qpid2d, kpid2d)


S_BLK = 256


def _expansion_matrix():
    r = lax.broadcasted_iota(jnp.int32, (DIM, 2 * DIM), 0)
    c = lax.broadcasted_iota(jnp.int32, (DIM, 2 * DIM), 1)
    cond_a = (r < HALF) & (c < DIM) & ((c >> 1) == r)
    cond_b = (r >= HALF) & (c >= DIM) & (((c - DIM) >> 1) == (r - HALF))
    sign = jnp.where((c & 1) == 1, 1.0, -1.0).astype(jnp.float32)
    return (jnp.where(cond_a, 1.0, 0.0) +
            jnp.where(cond_b, sign, 0.0)).astype(jnp.float32)


def _tc_apply(query, key, f3):
    def body(q_ref, k_ref, fq_ref, fk_ref, oq_ref, ok_ref):
        lane = lax.broadcasted_iota(jnp.int32, (S_BLK, B, NH, DIM), 3)
        swap_idx = lane ^ 1
        e_mat = _expansion_matrix()

        def apply(x_ref, f_ref, o_ref):
            x = x_ref[...]
            fc = f_ref[...].reshape(S_BLK * B, DIM)
            ab = lax.dot_general(fc, e_mat, (((1,), (0,)), ((), ())),
                                 precision=lax.Precision.HIGHEST,
                                 preferred_element_type=jnp.float32)
            a = ab[:, :DIM].reshape(S_BLK, B, 1, DIM)
            b = ab[:, DIM:].reshape(S_BLK, B, 1, DIM)
            xs = jnp.take_along_axis(x, swap_idx, axis=3)
            o_ref[...] = x * a + xs * b

        apply(q_ref, fq_ref, oq_ref)
        apply(k_ref, fk_ref, ok_ref)

    x_spec = pl.BlockSpec((S_BLK, B, NH, DIM), lambda i: (i, 0, 0, 0))
    fq_spec = pl.BlockSpec((1, S_BLK * B, DIM), lambda i: (0, i, 0))
    fk_spec = pl.BlockSpec((1, S_BLK * B, DIM), lambda i: (1, i, 0))
    return pl.pallas_call(
        body,
        grid=(SQ // S_BLK,),
        in_specs=[x_spec, x_spec, fq_spec, fk_spec],
        out_specs=[x_spec, x_spec],
        out_shape=[
            jax.ShapeDtypeStruct((SQ, B, NH, DIM), jnp.float32),
            jax.ShapeDtypeStruct((SQ, B, NH, DIM), jnp.float32),
        ],
        compiler_params=pltpu.CompilerParams(
            dimension_semantics=("parallel",),
            vmem_limit_bytes=120 * 1024 * 1024),
    )(query, key, f3, f3)


def kernel(query, key, query_position_ids, key_position_ids,
           freqs_cis_real, freqs_cis_imag):
    tab = jnp.concatenate([freqs_cis_real.astype(jnp.float32),
                           freqs_cis_imag.astype(jnp.float32)], axis=1)

    qpid2d = query_position_ids.astype(jnp.int32).reshape(ROWS // CHUNK, CHUNK)
    kpid2d = key_position_ids.astype(jnp.int32).reshape(ROWS // CHUNK, CHUNK)

    f = _sc_gather(tab, qpid2d, kpid2d)
    f3 = f.reshape(2, ROWS, DIM)

    q_out, k_out = _tc_apply(query.astype(jnp.float32),
                             key.astype(jnp.float32), f3)
    return q_out, k_out
